# initial kernel scaffold (unmeasured)
import jax
import jax.numpy as jnp
from jax import lax
from jax.experimental import pallas as pl
from jax.experimental.pallas import tpu as pltpu

N_DEV = 4
T = 1024
T_LOC = T // N_DEV
D = 1024
F = 2048
E = 16
E_LOC = E // N_DEV

_F32_MIN = jnp.finfo(jnp.float32).min


def kernel(x, router, W1, W2):
    router_t = router.T

    def body(x_ref, rt_ref, w1_hbm, w2_hbm, out_ref,
             xfull_ref, rtfull_ref, acc_ref,
             ag_buf, rtg_buf, rs_buf, w1_vmem, w2_vmem,
             ag_send, ag_recv, rtg_send, rtg_recv, rs_send, rs_recv,
             w1_sem, w2_sem):
        my_x = lax.axis_index("x")
        my_y = lax.axis_index("y")
        my_z = lax.axis_index("z")
        right = (my_x, (my_y + 1) % N_DEV, my_z)
        left = (my_x, (my_y + 3) % N_DEV, my_z)

        barrier = pltpu.get_barrier_semaphore()
        for nbr in (left, right):
            pl.semaphore_signal(barrier, inc=1, device_id=nbr,
                                device_id_type=pl.DeviceIdType.MESH)
        pl.semaphore_wait(barrier, 2)

        for s in (0, 1):
            pltpu.make_async_copy(w1_hbm.at[s], w1_vmem.at[s], w1_sem.at[s]).start()
            pltpu.make_async_copy(w2_hbm.at[s], w2_vmem.at[s], w2_sem.at[s]).start()

        xfull_ref[pl.ds(my_y * T_LOC, T_LOC), :] = x_ref[...]
        rtfull_ref[pl.ds(my_y * E_LOC, E_LOC), :] = rt_ref[...]

        for h in range(N_DEV - 1):
            x_src = x_ref if h == 0 else ag_buf.at[h - 1]
            r_src = rt_ref if h == 0 else rtg_buf.at[h - 1]
            x_rdma = pltpu.make_async_remote_copy(
                src_ref=x_src, dst_ref=ag_buf.at[h],
                send_sem=ag_send.at[h], recv_sem=ag_recv.at[h],
                device_id=right, device_id_type=pl.DeviceIdType.MESH)
            r_rdma = pltpu.make_async_remote_copy(
                src_ref=r_src, dst_ref=rtg_buf.at[h],
                send_sem=rtg_send.at[h], recv_sem=rtg_recv.at[h],
                device_id=right, device_id_type=pl.DeviceIdType.MESH)
            x_rdma.start()
            r_rdma.start()
            x_rdma.wait()
            r_rdma.wait()
            origin = (my_y + N_DEV - 1 - h) % N_DEV
            xfull_ref[pl.ds(origin * T_LOC, T_LOC), :] = ag_buf[h]
            rtfull_ref[pl.ds(origin * E_LOC, E_LOC), :] = rtg_buf[h]

        x_full = xfull_ref[...]
        gates = lax.dot_general(
            x_full, rtfull_ref[...], (((1,), (1,)), ((), ())),
            precision=lax.Precision.HIGHEST)
        ids = lax.broadcasted_iota(jnp.int32, (T, E), 1)
        m1 = jnp.max(gates, axis=1, keepdims=True)
        a1 = jnp.min(jnp.where(gates == m1, ids, E), axis=1, keepdims=True)
        g2 = jnp.where(ids == a1, _F32_MIN, gates)
        m2 = jnp.max(g2, axis=1, keepdims=True)
        a2 = jnp.min(jnp.where(g2 == m2, ids, E), axis=1, keepdims=True)
        r = jnp.exp(m2 - m1)
        s1 = 1.0 / (1.0 + r)
        s2 = r / (1.0 + r)

        acc = jnp.zeros((T, D), jnp.float32)
        for el in range(E_LOC):
            s = el % 2
            pltpu.make_async_copy(w1_hbm.at[el], w1_vmem.at[s], w1_sem.at[s]).wait()
            pltpu.make_async_copy(w2_hbm.at[el], w2_vmem.at[s], w2_sem.at[s]).wait()
            ge = my_y * E_LOC + el
            wt = (s1 * (a1 == ge).astype(jnp.float32)
                  + s2 * (a2 == ge).astype(jnp.float32))
            h1 = jnp.maximum(
                jnp.dot(x_full, w1_vmem[s], preferred_element_type=jnp.float32),
                0.0)
            ye = jnp.dot(h1, w2_vmem[s], preferred_element_type=jnp.float32)
            acc = acc + ye * wt
            if el + 2 < E_LOC:
                pltpu.make_async_copy(
                    w1_hbm.at[el + 2], w1_vmem.at[s], w1_sem.at[s]).start()
                pltpu.make_async_copy(
                    w2_hbm.at[el + 2], w2_vmem.at[s], w2_sem.at[s]).start()
        acc_ref[...] = acc

        for t in range(N_DEV - 1):
            c_send = (my_y + N_DEV - 1 - t) % N_DEV
            if t == 0:
                src = acc_ref.at[pl.ds(c_send * T_LOC, T_LOC), :]
            else:
                rs_buf[t - 1, :, :] = (
                    rs_buf[t - 1] + acc_ref[pl.ds(c_send * T_LOC, T_LOC), :])
                src = rs_buf.at[t - 1]
            rdma = pltpu.make_async_remote_copy(
                src_ref=src, dst_ref=rs_buf.at[t],
                send_sem=rs_send.at[t], recv_sem=rs_recv.at[t],
                device_id=right, device_id_type=pl.DeviceIdType.MESH)
            rdma.start()
            rdma.wait()
        out_ref[...] = rs_buf[N_DEV - 2] + acc_ref[pl.ds(my_y * T_LOC, T_LOC), :]

    return pl.pallas_call(
        body,
        out_shape=jax.ShapeDtypeStruct((T_LOC, D), jnp.float32),
        in_specs=[
            pl.BlockSpec(memory_space=pltpu.VMEM),
            pl.BlockSpec(memory_space=pltpu.VMEM),
            pl.BlockSpec(memory_space=pltpu.ANY),
            pl.BlockSpec(memory_space=pltpu.ANY),
        ],
        out_specs=pl.BlockSpec(memory_space=pltpu.VMEM),
        scratch_shapes=[
            pltpu.VMEM((T, D), jnp.float32),
            pltpu.VMEM((E, D), jnp.float32),
            pltpu.VMEM((T, D), jnp.float32),
            pltpu.VMEM((N_DEV - 1, T_LOC, D), jnp.float32),
            pltpu.VMEM((N_DEV - 1, E_LOC, D), jnp.float32),
            pltpu.VMEM((N_DEV - 1, T_LOC, D), jnp.float32),
            pltpu.VMEM((2, D, F), jnp.float32),
            pltpu.VMEM((2, F, D), jnp.float32),
            pltpu.SemaphoreType.DMA((N_DEV - 1,)),
            pltpu.SemaphoreType.DMA((N_DEV - 1,)),
            pltpu.SemaphoreType.DMA((N_DEV - 1,)),
            pltpu.SemaphoreType.DMA((N_DEV - 1,)),
            pltpu.SemaphoreType.DMA((N_DEV - 1,)),
            pltpu.SemaphoreType.DMA((N_DEV - 1,)),
            pltpu.SemaphoreType.DMA((2,)),
            pltpu.SemaphoreType.DMA((2,)),
        ],
        compiler_params=pltpu.CompilerParams(collective_id=0),
    )(x, router_t, W1, W2)


# baseline (device time: 136548 ns/iter reference)
import jax
import jax.numpy as jnp
from jax import lax
from jax.experimental import pallas as pl
from jax.experimental.pallas import tpu as pltpu

N_DEV = 4
T = 1024
T_LOC = T // N_DEV
D = 1024
F = 2048
E = 16
E_LOC = E // N_DEV

_F32_MIN = jnp.finfo(jnp.float32).min


def kernel(x, router, W1, W2):
    router_t = router.T

    def body(x_ref, rt_ref, w1_hbm, w2_hbm, out_ref,
             xfull_ref, rtfull_ref, acc_ref,
             ag_buf, rtg_buf, rs_buf, w1_vmem, w2_vmem,
             ag_send, ag_recv, rtg_send, rtg_recv, rs_send, rs_recv,
             w1_sem, w2_sem):
        my_x = lax.axis_index("x")
        my_y = lax.axis_index("y")
        my_z = lax.axis_index("z")
        right = (my_x, (my_y + 1) % N_DEV, my_z)
        left = (my_x, (my_y + 3) % N_DEV, my_z)

        barrier = pltpu.get_barrier_semaphore()
        for nbr in (left, right):
            pl.semaphore_signal(barrier, inc=1, device_id=nbr,
                                device_id_type=pl.DeviceIdType.MESH)
        pl.semaphore_wait(barrier, 2)

        for s in (0, 1):
            pltpu.make_async_copy(w1_hbm.at[s], w1_vmem.at[s], w1_sem.at[s]).start()
            pltpu.make_async_copy(w2_hbm.at[s], w2_vmem.at[s], w2_sem.at[s]).start()

        xfull_ref[pl.ds(my_y * T_LOC, T_LOC), :] = x_ref[...]
        rtfull_ref[pl.ds(my_y, 1)] = rt_ref[...][None]

        for h in range(N_DEV - 1):
            x_src = x_ref if h == 0 else ag_buf.at[h - 1]
            r_src = rt_ref if h == 0 else rtg_buf.at[h - 1]
            x_rdma = pltpu.make_async_remote_copy(
                src_ref=x_src, dst_ref=ag_buf.at[h],
                send_sem=ag_send.at[h], recv_sem=ag_recv.at[h],
                device_id=right, device_id_type=pl.DeviceIdType.MESH)
            r_rdma = pltpu.make_async_remote_copy(
                src_ref=r_src, dst_ref=rtg_buf.at[h],
                send_sem=rtg_send.at[h], recv_sem=rtg_recv.at[h],
                device_id=right, device_id_type=pl.DeviceIdType.MESH)
            x_rdma.start()
            r_rdma.start()
            x_rdma.wait()
            r_rdma.wait()
            origin = (my_y + N_DEV - 1 - h) % N_DEV
            xfull_ref[pl.ds(origin * T_LOC, T_LOC), :] = ag_buf[h]
            rtfull_ref[pl.ds(origin, 1)] = rtg_buf[h][None]

        rt_all = rtfull_ref[...].reshape(E, D)
        gates = lax.dot_general(
            xfull_ref[...], rt_all, (((1,), (1,)), ((), ())),
            precision=lax.Precision.HIGHEST)
        ids = lax.broadcasted_iota(jnp.int32, (T, E), 1)
        m1 = jnp.max(gates, axis=1, keepdims=True)
        a1 = jnp.min(jnp.where(gates == m1, ids, E), axis=1, keepdims=True)
        g2 = jnp.where(ids == a1, _F32_MIN, gates)
        m2 = jnp.max(g2, axis=1, keepdims=True)
        a2 = jnp.min(jnp.where(g2 == m2, ids, E), axis=1, keepdims=True)
        r = jnp.exp(m2 - m1)
        s1 = 1.0 / (1.0 + r)
        s2 = r / (1.0 + r)

        for el in range(E_LOC):
            s = el % 2
            pltpu.make_async_copy(w1_hbm.at[el], w1_vmem.at[s], w1_sem.at[s]).wait()
            pltpu.make_async_copy(w2_hbm.at[el], w2_vmem.at[s], w2_sem.at[s]).wait()
            ge = my_y * E_LOC + el
            wt = (s1 * (a1 == ge).astype(jnp.float32)
                  + s2 * (a2 == ge).astype(jnp.float32))
            for c in range(N_DEV):
                xc = xfull_ref[c * T_LOC:(c + 1) * T_LOC, :]
                h1 = jnp.maximum(
                    jnp.dot(xc, w1_vmem[s], preferred_element_type=jnp.float32),
                    0.0)
                ye = jnp.dot(h1, w2_vmem[s], preferred_element_type=jnp.float32)
                contrib = ye * wt[c * T_LOC:(c + 1) * T_LOC]
                if el == 0:
                    acc_ref[c * T_LOC:(c + 1) * T_LOC, :] = contrib
                else:
                    acc_ref[c * T_LOC:(c + 1) * T_LOC, :] = (
                        acc_ref[c * T_LOC:(c + 1) * T_LOC, :] + contrib)
            if el + 2 < E_LOC:
                pltpu.make_async_copy(
                    w1_hbm.at[el + 2], w1_vmem.at[s], w1_sem.at[s]).start()
                pltpu.make_async_copy(
                    w2_hbm.at[el + 2], w2_vmem.at[s], w2_sem.at[s]).start()

        for t in range(N_DEV - 1):
            c_send = (my_y + N_DEV - 1 - t) % N_DEV
            if t == 0:
                src = acc_ref.at[pl.ds(c_send * T_LOC, T_LOC), :]
            else:
                rs_buf[t - 1, :, :] = (
                    rs_buf[t - 1] + acc_ref[pl.ds(c_send * T_LOC, T_LOC), :])
                src = rs_buf.at[t - 1]
            rdma = pltpu.make_async_remote_copy(
                src_ref=src, dst_ref=rs_buf.at[t],
                send_sem=rs_send.at[t], recv_sem=rs_recv.at[t],
                device_id=right, device_id_type=pl.DeviceIdType.MESH)
            rdma.start()
            rdma.wait()
        out_ref[...] = rs_buf[N_DEV - 2] + acc_ref[pl.ds(my_y * T_LOC, T_LOC), :]

    return pl.pallas_call(
        body,
        out_shape=jax.ShapeDtypeStruct((T_LOC, D), jnp.float32),
        in_specs=[
            pl.BlockSpec(memory_space=pltpu.VMEM),
            pl.BlockSpec(memory_space=pltpu.VMEM),
            pl.BlockSpec(memory_space=pltpu.MemorySpace.HBM),
            pl.BlockSpec(memory_space=pltpu.MemorySpace.HBM),
        ],
        out_specs=pl.BlockSpec(memory_space=pltpu.VMEM),
        scratch_shapes=[
            pltpu.VMEM((T, D), jnp.float32),
            pltpu.VMEM((N_DEV, E_LOC, D), jnp.float32),
            pltpu.VMEM((T, D), jnp.float32),
            pltpu.VMEM((N_DEV - 1, T_LOC, D), jnp.float32),
            pltpu.VMEM((N_DEV - 1, E_LOC, D), jnp.float32),
            pltpu.VMEM((N_DEV - 1, T_LOC, D), jnp.float32),
            pltpu.VMEM((2, D, F), jnp.float32),
            pltpu.VMEM((2, F, D), jnp.float32),
            pltpu.SemaphoreType.DMA((N_DEV - 1,)),
            pltpu.SemaphoreType.DMA((N_DEV - 1,)),
            pltpu.SemaphoreType.DMA((N_DEV - 1,)),
            pltpu.SemaphoreType.DMA((N_DEV - 1,)),
            pltpu.SemaphoreType.DMA((N_DEV - 1,)),
            pltpu.SemaphoreType.DMA((N_DEV - 1,)),
            pltpu.SemaphoreType.DMA((2,)),
            pltpu.SemaphoreType.DMA((2,)),
        ],
        compiler_params=pltpu.CompilerParams(
            collective_id=0, vmem_limit_bytes=64 * 1024 * 1024),
    )(x, router_t, W1, W2)


# device time: 120601 ns/iter; 1.1322x vs baseline; 1.1322x over previous
import jax
import jax.numpy as jnp
from jax import lax
from jax.experimental import pallas as pl
from jax.experimental.pallas import tpu as pltpu

N_DEV = 4
T = 1024
T_LOC = T // N_DEV
D = 1024
F = 2048
E = 16
E_LOC = E // N_DEV

_F32_MIN = jnp.finfo(jnp.float32).min


def kernel(x, router, W1, W2):
    router_t = router.T

    def body(x_ref, rt_ref, w1_hbm, w2_hbm, out_ref,
             xfull_ref, rtfull_ref, acc_ref,
             ag_buf, rtg_buf, rs_buf, w1_vmem, w2_vmem,
             ag_send, ag_recv, rtg_send, rtg_recv, rs_send, rs_recv,
             w1_sem, w2_sem):
        my_x = lax.axis_index("x")
        my_y = lax.axis_index("y")
        my_z = lax.axis_index("z")
        right = (my_x, (my_y + 1) % N_DEV, my_z)
        left = (my_x, (my_y + 3) % N_DEV, my_z)

        barrier = pltpu.get_barrier_semaphore()
        for nbr in (left, right):
            pl.semaphore_signal(barrier, inc=1, device_id=nbr,
                                device_id_type=pl.DeviceIdType.MESH)
        pl.semaphore_wait(barrier, 2)

        def w_copies(el, s):
            return (pltpu.make_async_copy(w1_hbm.at[el], w1_vmem.at[s],
                                          w1_sem.at[s]),
                    pltpu.make_async_copy(w2_hbm.at[el], w2_vmem.at[s],
                                          w2_sem.at[s]))

        for el in (0, 1):
            for c in w_copies(el, el):
                c.start()

        def x_hop(h):
            return pltpu.make_async_remote_copy(
                src_ref=x_ref if h == 0 else ag_buf.at[h - 1],
                dst_ref=ag_buf.at[h],
                send_sem=ag_send.at[h], recv_sem=ag_recv.at[h],
                device_id=right, device_id_type=pl.DeviceIdType.MESH)

        hop0 = x_hop(0)
        hop0.start()
        hops = [hop0, x_hop(1), x_hop(2)]

        xfull_ref[pl.ds(my_y * T_LOC, T_LOC), :] = x_ref[...]
        rtfull_ref[pl.ds(my_y, 1)] = rt_ref[...][None]

        for h in range(N_DEV - 1):
            r_rdma = pltpu.make_async_remote_copy(
                src_ref=rt_ref if h == 0 else rtg_buf.at[h - 1],
                dst_ref=rtg_buf.at[h],
                send_sem=rtg_send.at[h], recv_sem=rtg_recv.at[h],
                device_id=right, device_id_type=pl.DeviceIdType.MESH)
            r_rdma.start()
            r_rdma.wait()
            origin = (my_y + N_DEV - 1 - h) % N_DEV
            rtfull_ref[pl.ds(origin, 1)] = rtg_buf[h][None]

        rt_all = rtfull_ref[...].reshape(E, D)

        def do_route(xc):
            gates = lax.dot_general(
                xc, rt_all, (((1,), (1,)), ((), ())),
                precision=lax.Precision.HIGHEST)
            ids = lax.broadcasted_iota(jnp.int32, (T_LOC, E), 1)
            m1 = jnp.max(gates, axis=1, keepdims=True)
            a1 = jnp.min(jnp.where(gates == m1, ids, E), axis=1, keepdims=True)
            g2 = jnp.where(ids == a1, _F32_MIN, gates)
            m2 = jnp.max(g2, axis=1, keepdims=True)
            a2 = jnp.min(jnp.where(g2 == m2, ids, E), axis=1, keepdims=True)
            r = jnp.exp(m2 - m1)
            return a1, a2, 1.0 / (1.0 + r), r / (1.0 + r)

        def expert_chunk(el, s, xc, origin, rinfo, first):
            a1, a2, s1, s2 = rinfo
            ge = my_y * E_LOC + el
            wt = (s1 * (a1 == ge).astype(jnp.float32)
                  + s2 * (a2 == ge).astype(jnp.float32))
            h1 = jnp.maximum(
                jnp.dot(xc, w1_vmem[s], preferred_element_type=jnp.float32),
                0.0)
            contrib = jnp.dot(
                h1, w2_vmem[s], preferred_element_type=jnp.float32) * wt
            if first:
                acc_ref[pl.ds(origin * T_LOC, T_LOC), :] = contrib
            else:
                acc_ref[pl.ds(origin * T_LOC, T_LOC), :] = (
                    acc_ref[pl.ds(origin * T_LOC, T_LOC), :] + contrib)

        for el in (0, 1):
            for c in w_copies(el, el):
                c.wait()

        route = [None] * N_DEV
        for k in range(N_DEV):
            if k == 0:
                xc = x_ref[...]
            else:
                hops[k - 1].wait()
                if k < N_DEV - 1:
                    hops[k].start()
                xc = ag_buf[k - 1]
                origin = (my_y + N_DEV - k) % N_DEV
                xfull_ref[pl.ds(origin * T_LOC, T_LOC), :] = xc
            origin = (my_y + N_DEV - k) % N_DEV
            route[k] = do_route(xc)
            expert_chunk(0, 0, xc, origin, route[k], first=True)
            if k == N_DEV - 1:
                for c in w_copies(2, 0):
                    c.start()
            expert_chunk(1, 1, xc, origin, route[k], first=False)
            if k == N_DEV - 1:
                for c in w_copies(3, 1):
                    c.start()

        for c in w_copies(2, 0):
            c.wait()
        for k in (1, 2, 3, 0):
            origin = (my_y + N_DEV - k) % N_DEV
            xc = xfull_ref[pl.ds(origin * T_LOC, T_LOC), :]
            expert_chunk(2, 0, xc, origin, route[k], first=False)

        for c in w_copies(3, 1):
            c.wait()
        rs = [None] * (N_DEV - 1)
        for t in range(N_DEV - 1):
            k = t + 1
            origin = (my_y + N_DEV - k) % N_DEV
            xc = xfull_ref[pl.ds(origin * T_LOC, T_LOC), :]
            expert_chunk(3, 1, xc, origin, route[k], first=False)
            if t == 0:
                src = acc_ref.at[pl.ds(origin * T_LOC, T_LOC), :]
            else:
                rs[t - 1].wait_recv()
                rs_buf[t - 1, :, :] = (
                    rs_buf[t - 1] + acc_ref[pl.ds(origin * T_LOC, T_LOC), :])
                src = rs_buf.at[t - 1]
            rs[t] = pltpu.make_async_remote_copy(
                src_ref=src, dst_ref=rs_buf.at[t],
                send_sem=rs_send.at[t], recv_sem=rs_recv.at[t],
                device_id=right, device_id_type=pl.DeviceIdType.MESH)
            rs[t].start()
        expert_chunk(3, 1, x_ref[...], my_y, route[0], first=False)
        rs[N_DEV - 2].wait_recv()
        out_ref[...] = rs_buf[N_DEV - 2] + acc_ref[pl.ds(my_y * T_LOC, T_LOC), :]
        for t in range(N_DEV - 1):
            rs[t].wait_send()

    return pl.pallas_call(
        body,
        out_shape=jax.ShapeDtypeStruct((T_LOC, D), jnp.float32),
        in_specs=[
            pl.BlockSpec(memory_space=pltpu.VMEM),
            pl.BlockSpec(memory_space=pltpu.VMEM),
            pl.BlockSpec(memory_space=pltpu.MemorySpace.HBM),
            pl.BlockSpec(memory_space=pltpu.MemorySpace.HBM),
        ],
        out_specs=pl.BlockSpec(memory_space=pltpu.VMEM),
        scratch_shapes=[
            pltpu.VMEM((T, D), jnp.float32),
            pltpu.VMEM((N_DEV, E_LOC, D), jnp.float32),
            pltpu.VMEM((T, D), jnp.float32),
            pltpu.VMEM((N_DEV - 1, T_LOC, D), jnp.float32),
            pltpu.VMEM((N_DEV - 1, E_LOC, D), jnp.float32),
            pltpu.VMEM((N_DEV - 1, T_LOC, D), jnp.float32),
            pltpu.VMEM((2, D, F), jnp.float32),
            pltpu.VMEM((2, F, D), jnp.float32),
            pltpu.SemaphoreType.DMA((N_DEV - 1,)),
            pltpu.SemaphoreType.DMA((N_DEV - 1,)),
            pltpu.SemaphoreType.DMA((N_DEV - 1,)),
            pltpu.SemaphoreType.DMA((N_DEV - 1,)),
            pltpu.SemaphoreType.DMA((N_DEV - 1,)),
            pltpu.SemaphoreType.DMA((N_DEV - 1,)),
            pltpu.SemaphoreType.DMA((2,)),
            pltpu.SemaphoreType.DMA((2,)),
        ],
        compiler_params=pltpu.CompilerParams(
            collective_id=0, vmem_limit_bytes=64 * 1024 * 1024),
    )(x, router_t, W1, W2)


# device time: 113894 ns/iter; 1.1989x vs baseline; 1.0589x over previous
import jax
import jax.numpy as jnp
from jax import lax
from jax.experimental import pallas as pl
from jax.experimental.pallas import tpu as pltpu

N_DEV = 4
T = 1024
T_LOC = T // N_DEV
D = 1024
F = 2048
E = 16
E_LOC = E // N_DEV

_F32_MIN = jnp.finfo(jnp.float32).min


def kernel(x, router, W1, W2):
    router_t = router.T

    def body(x_ref, rt_ref, w1_hbm, w2_hbm, out_ref,
             xfull_ref, rtfull_ref, acc_ref,
             ag_buf, rtg_buf, rs_buf, w1_vmem, w2_vmem,
             ag_send, ag_recv, rtg_send, rtg_recv, rs_send, rs_recv,
             w1_sem, w2_sem):
        my_x = lax.axis_index("x")
        my_y = lax.axis_index("y")
        my_z = lax.axis_index("z")
        right = (my_x, (my_y + 1) % N_DEV, my_z)
        left = (my_x, (my_y + 3) % N_DEV, my_z)

        barrier = pltpu.get_barrier_semaphore()
        for nbr in (left, right):
            pl.semaphore_signal(barrier, inc=1, device_id=nbr,
                                device_id_type=pl.DeviceIdType.MESH)
        pl.semaphore_wait(barrier, 2)

        def w_copies(el, s):
            return (pltpu.make_async_copy(w1_hbm.at[el], w1_vmem.at[s],
                                          w1_sem.at[s]),
                    pltpu.make_async_copy(w2_hbm.at[el], w2_vmem.at[s],
                                          w2_sem.at[s]))

        for el in (0, 1):
            for c in w_copies(el, el):
                c.start()

        def x_hop(h):
            return pltpu.make_async_remote_copy(
                src_ref=x_ref if h == 0 else ag_buf.at[h - 1],
                dst_ref=ag_buf.at[h],
                send_sem=ag_send.at[h], recv_sem=ag_recv.at[h],
                device_id=right, device_id_type=pl.DeviceIdType.MESH)

        hop0 = x_hop(0)
        hop0.start()
        hops = [hop0, x_hop(1), x_hop(2)]

        xfull_ref[pl.ds(my_y * T_LOC, T_LOC), :] = x_ref[...]
        rtfull_ref[pl.ds(my_y, 1)] = rt_ref[...][None]

        for h in range(N_DEV - 1):
            r_rdma = pltpu.make_async_remote_copy(
                src_ref=rt_ref if h == 0 else rtg_buf.at[h - 1],
                dst_ref=rtg_buf.at[h],
                send_sem=rtg_send.at[h], recv_sem=rtg_recv.at[h],
                device_id=right, device_id_type=pl.DeviceIdType.MESH)
            r_rdma.start()
            r_rdma.wait()
            origin = (my_y + N_DEV - 1 - h) % N_DEV
            rtfull_ref[pl.ds(origin, 1)] = rtg_buf[h][None]

        rt_all = rtfull_ref[...].reshape(E, D)

        def do_route(xc):
            gates = lax.dot_general(
                xc, rt_all, (((1,), (1,)), ((), ())),
                precision=lax.Precision.HIGHEST)
            ids = lax.broadcasted_iota(jnp.int32, (T_LOC, E), 1)
            m1 = jnp.max(gates, axis=1, keepdims=True)
            a1 = jnp.min(jnp.where(gates == m1, ids, E), axis=1, keepdims=True)
            g2 = jnp.where(ids == a1, _F32_MIN, gates)
            m2 = jnp.max(g2, axis=1, keepdims=True)
            a2 = jnp.min(jnp.where(g2 == m2, ids, E), axis=1, keepdims=True)
            r = jnp.exp(m2 - m1)
            return a1, a2, 1.0 / (1.0 + r), r / (1.0 + r)

        def expert_chunk(el, s, xc, origin, rinfo, first):
            a1, a2, s1, s2 = rinfo
            ge = my_y * E_LOC + el
            wt = (s1 * (a1 == ge).astype(jnp.float32)
                  + s2 * (a2 == ge).astype(jnp.float32))
            h1 = jnp.maximum(
                jnp.dot(xc, w1_vmem[s], preferred_element_type=jnp.float32),
                0.0)
            contrib = jnp.dot(
                h1, w2_vmem[s], preferred_element_type=jnp.float32) * wt
            if first:
                acc_ref[pl.ds(origin * T_LOC, T_LOC), :] = contrib
            else:
                acc_ref[pl.ds(origin * T_LOC, T_LOC), :] = (
                    acc_ref[pl.ds(origin * T_LOC, T_LOC), :] + contrib)

        for el in (0, 1):
            for c in w_copies(el, el):
                c.wait()

        route = [None] * N_DEV
        for k in range(N_DEV):
            if k == 0:
                xc = x_ref[...]
            else:
                hops[k - 1].wait()
                if k < N_DEV - 1:
                    hops[k].start()
                xc = ag_buf[k - 1]
                origin = (my_y + N_DEV - k) % N_DEV
                xfull_ref[pl.ds(origin * T_LOC, T_LOC), :] = xc
            origin = (my_y + N_DEV - k) % N_DEV
            route[k] = do_route(xc)
            expert_chunk(0, 0, xc, origin, route[k], first=True)
            if k == N_DEV - 1:
                for c in w_copies(2, 0):
                    c.start()
            expert_chunk(1, 1, xc, origin, route[k], first=False)
            if k == N_DEV - 1:
                for c in w_copies(3, 1):
                    c.start()

        for c in w_copies(2, 0):
            c.wait()
        for k in (1, 2, 3, 0):
            origin = (my_y + N_DEV - k) % N_DEV
            xc = xfull_ref[pl.ds(origin * T_LOC, T_LOC), :]
            expert_chunk(2, 0, xc, origin, route[k], first=False)

        for c in w_copies(3, 1):
            c.wait()
        H = T_LOC // 2

        def expert3_half(origin, rinfo, hs):
            a1, a2, s1, s2 = rinfo
            lo, hi = hs * H, (hs + 1) * H
            ge = my_y * E_LOC + 3
            wt = (s1[lo:hi] * (a1[lo:hi] == ge).astype(jnp.float32)
                  + s2[lo:hi] * (a2[lo:hi] == ge).astype(jnp.float32))
            xc = xfull_ref[pl.ds(origin * T_LOC + lo, H), :]
            h1 = jnp.maximum(
                jnp.dot(xc, w1_vmem[1], preferred_element_type=jnp.float32),
                0.0)
            contrib = jnp.dot(
                h1, w2_vmem[1], preferred_element_type=jnp.float32) * wt
            acc_ref[pl.ds(origin * T_LOC + lo, H), :] = (
                acc_ref[pl.ds(origin * T_LOC + lo, H), :] + contrib)

        rs = [[None, None] for _ in range(N_DEV - 1)]
        for t in range(N_DEV - 1):
            k = t + 1
            origin = (my_y + N_DEV - k) % N_DEV
            for hs in (0, 1):
                lo = hs * H
                expert3_half(origin, route[k], hs)
                if t == 0:
                    src = acc_ref.at[pl.ds(origin * T_LOC + lo, H), :]
                else:
                    rs[t - 1][hs].wait_recv()
                    rs_buf[t - 1, lo:lo + H, :] = (
                        rs_buf[t - 1, lo:lo + H, :]
                        + acc_ref[pl.ds(origin * T_LOC + lo, H), :])
                    src = rs_buf.at[t - 1, pl.ds(lo, H), :]
                rs[t][hs] = pltpu.make_async_remote_copy(
                    src_ref=src, dst_ref=rs_buf.at[t, pl.ds(lo, H), :],
                    send_sem=rs_send.at[t, hs], recv_sem=rs_recv.at[t, hs],
                    device_id=right, device_id_type=pl.DeviceIdType.MESH)
                rs[t][hs].start()
        for hs in (0, 1):
            expert3_half(my_y, route[0], hs)
        for hs in (0, 1):
            rs[N_DEV - 2][hs].wait_recv()
        out_ref[...] = rs_buf[N_DEV - 2] + acc_ref[pl.ds(my_y * T_LOC, T_LOC), :]
        for t in range(N_DEV - 1):
            for hs in (0, 1):
                rs[t][hs].wait_send()

    return pl.pallas_call(
        body,
        out_shape=jax.ShapeDtypeStruct((T_LOC, D), jnp.float32),
        in_specs=[
            pl.BlockSpec(memory_space=pltpu.VMEM),
            pl.BlockSpec(memory_space=pltpu.VMEM),
            pl.BlockSpec(memory_space=pltpu.MemorySpace.HBM),
            pl.BlockSpec(memory_space=pltpu.MemorySpace.HBM),
        ],
        out_specs=pl.BlockSpec(memory_space=pltpu.VMEM),
        scratch_shapes=[
            pltpu.VMEM((T, D), jnp.float32),
            pltpu.VMEM((N_DEV, E_LOC, D), jnp.float32),
            pltpu.VMEM((T, D), jnp.float32),
            pltpu.VMEM((N_DEV - 1, T_LOC, D), jnp.float32),
            pltpu.VMEM((N_DEV - 1, E_LOC, D), jnp.float32),
            pltpu.VMEM((N_DEV - 1, T_LOC, D), jnp.float32),
            pltpu.VMEM((2, D, F), jnp.float32),
            pltpu.VMEM((2, F, D), jnp.float32),
            pltpu.SemaphoreType.DMA((N_DEV - 1,)),
            pltpu.SemaphoreType.DMA((N_DEV - 1,)),
            pltpu.SemaphoreType.DMA((N_DEV - 1,)),
            pltpu.SemaphoreType.DMA((N_DEV - 1,)),
            pltpu.SemaphoreType.DMA((N_DEV - 1, 2)),
            pltpu.SemaphoreType.DMA((N_DEV - 1, 2)),
            pltpu.SemaphoreType.DMA((2,)),
            pltpu.SemaphoreType.DMA((2,)),
        ],
        compiler_params=pltpu.CompilerParams(
            collective_id=0, vmem_limit_bytes=64 * 1024 * 1024),
    )(x, router_t, W1, W2)
